# baseline (speedup 1.0000x reference)
import functools

import jax
import jax.numpy as jnp
from jax import lax
from jax.experimental import pallas as pl
from jax.experimental.pallas import tpu as pltpu
from jax.experimental.pallas import tpu_sc as plsc

D = 64
B = 16384

NC = 2
NS = 16
NW = NC * NS
B_PER_W = B // NW
CHUNK = 128
NCHUNK = B_PER_W // CHUNK
GROUPS_PER_CHUNK = CHUNK // 16
ROWS2D = B // CHUNK

_LN2 = 0.6931471805599453
_SQRT_MAGIC = 0x1FBD1DF5


def _sc_kernel_body(user_table, item_table, user_bias, item_bias, gb16,
                    uids2d, iids2d, out,
                    uidx_v, iidx_v, u_rows, v_rows, ub_v, ib_v,
                    o_v, gb_v, s0, s1, s2, s3):
    wid = lax.axis_index("s") * NC + lax.axis_index("c")
    row0 = wid * NCHUNK
    sems = [s0, s1, s2, s3]

    pltpu.sync_copy(gb16, gb_v)
    pltpu.sync_copy(uids2d.at[pl.ds(row0, NCHUNK)], uidx_v)
    pltpu.sync_copy(iids2d.at[pl.ds(row0, NCHUNK)], iidx_v)

    handles = []
    for c in range(NCHUNK):
        sem = sems[c]
        handles.append((
            pltpu.async_copy(user_table.at[uidx_v.at[c]],
                             u_rows.at[pl.ds(c * CHUNK, CHUNK)], sem),
            pltpu.async_copy(item_table.at[iidx_v.at[c]],
                             v_rows.at[pl.ds(c * CHUNK, CHUNK)], sem),
            pltpu.async_copy(user_bias.at[uidx_v.at[c]], ub_v.at[c], sem),
            pltpu.async_copy(item_bias.at[iidx_v.at[c]], ib_v.at[c], sem),
        ))

    lanes = lax.iota(jnp.int32, 16)
    ix8 = lax.bitwise_xor(lanes, 8)
    ix4 = lax.bitwise_xor(lanes, 4)
    ix2 = lax.bitwise_xor(lanes, 2)
    ix1 = lax.bitwise_xor(lanes, 1)
    in_lo4 = lanes < 4
    in_lo8 = lanes < 8
    gb = gb_v[...]

    def shuf(v, ix):
        return v.at[ix].get(mode="promise_in_bounds")

    def f32bits(v):
        return lax.bitcast_convert_type(v, jnp.int32)

    def bits32f(v):
        return lax.bitcast_convert_type(v, jnp.float32)

    for c in range(NCHUNK):
        for h in handles[c]:
            h.wait()

        def group_body(j, _, c=c):
            uu = jnp.zeros((16,), jnp.float32)
            vv = jnp.zeros((16,), jnp.float32)
            dd = jnp.zeros((16,), jnp.float32)
            for jj in range(16):
                r = c * CHUNK + j * 16 + jj
                uu_p = jnp.zeros((16,), jnp.float32)
                vv_p = jnp.zeros((16,), jnp.float32)
                dd_p = jnp.zeros((16,), jnp.float32)
                for k in range(D // 16):
                    uk = u_rows[r, pl.ds(k * 16, 16)]
                    vk = v_rows[r, pl.ds(k * 16, 16)]
                    uu_p = uu_p + uk * uk
                    vv_p = vv_p + vk * vk
                    dk = uk - vk
                    dd_p = dd_p + dk * dk
                uu_p = uu_p + shuf(uu_p, ix8)
                vv_p = vv_p + shuf(vv_p, ix8)
                dd_p = dd_p + shuf(dd_p, ix8)
                uu_p = uu_p + shuf(uu_p, ix4)
                vv_p = vv_p + shuf(vv_p, ix4)
                dd_p = dd_p + shuf(dd_p, ix4)
                w = jnp.where(in_lo4, uu_p, jnp.where(in_lo8, vv_p, dd_p))
                w = w + shuf(w, ix2)
                w = w + shuf(w, ix1)
                m = lanes == jj
                uu = jnp.where(m, w[0], uu)
                vv = jnp.where(m, w[4], vv)
                dd = jnp.where(m, w[8], dd)
            den = (1.0 - uu) * (1.0 - vv) + 1e-6
            t = (2.0 * dd) / den
            x = 1.0 + t
            y = t * (t + 2.0)
            g = bits32f(lax.shift_right_logical(f32bits(y), 1)
                        + _SQRT_MAGIC)
            for _newton in range(4):
                g = 0.5 * (g + y / g)
            z = x + g
            zb = f32bits(z)
            e = lax.convert_element_type(
                lax.shift_right_logical(zb, 23) - 127, jnp.float32)
            mant = bits32f(lax.bitwise_or(
                lax.bitwise_and(zb, 0x007FFFFF), 0x3F800000))
            wq = (mant - 1.0) / (mant + 1.0)
            w2 = wq * wq
            p = jnp.full((16,), 1.0 / 9.0, jnp.float32)
            for coef in (1.0 / 7.0, 1.0 / 5.0, 1.0 / 3.0, 1.0):
                p = p * w2 + coef
            dist = e * _LN2 + 2.0 * wq * p
            ubv = ub_v[c, pl.ds(j * 16, 16)]
            ibv = ib_v[c, pl.ds(j * 16, 16)]
            o_v[c, pl.ds(j * 16, 16)] = ubv + ibv + gb - dist
            return _

        lax.fori_loop(0, GROUPS_PER_CHUNK, group_body, None)
        pltpu.sync_copy(o_v.at[pl.ds(c, 1)], out.at[pl.ds(row0 + c, 1)])


@jax.jit
def _sc_stage(uids2d, iids2d, user_table, item_table, user_bias, item_bias,
              gb16):
    mesh = plsc.VectorSubcoreMesh(core_axis_name="c", subcore_axis_name="s")
    f = functools.partial(
        pl.kernel,
        mesh=mesh,
        out_type=jax.ShapeDtypeStruct((ROWS2D, CHUNK), jnp.float32),
        scratch_types=[
            pltpu.VMEM((NCHUNK, CHUNK), jnp.int32),
            pltpu.VMEM((NCHUNK, CHUNK), jnp.int32),
            pltpu.VMEM((B_PER_W, D), jnp.float32),
            pltpu.VMEM((B_PER_W, D), jnp.float32),
            pltpu.VMEM((NCHUNK, CHUNK), jnp.float32),
            pltpu.VMEM((NCHUNK, CHUNK), jnp.float32),
            pltpu.VMEM((NCHUNK, CHUNK), jnp.float32),
            pltpu.VMEM((16,), jnp.float32),
            pltpu.SemaphoreType.DMA,
            pltpu.SemaphoreType.DMA,
            pltpu.SemaphoreType.DMA,
            pltpu.SemaphoreType.DMA,
        ],
        compiler_params=pltpu.CompilerParams(use_tc_tiling_on_sc=False),
    )(_sc_kernel_body)
    return f(user_table, item_table, user_bias, item_bias, gb16,
             uids2d, iids2d)


@jax.jit
def kernel(user_ids, item_ids, user_table, item_table, user_bias_table,
           item_bias_table, global_bias):
    uids2d = user_ids.reshape(ROWS2D, CHUNK)
    iids2d = item_ids.reshape(ROWS2D, CHUNK)
    gb16 = jnp.broadcast_to(global_bias, (16,))
    out2d = _sc_stage(uids2d, iids2d, user_table, item_table,
                      user_bias_table.reshape(-1),
                      item_bias_table.reshape(-1), gb16)
    return out2d.reshape(B)


# speedup vs baseline: 1.0037x; 1.0037x over previous
import functools

import jax
import jax.numpy as jnp
from jax import lax
from jax.experimental import pallas as pl
from jax.experimental.pallas import tpu as pltpu
from jax.experimental.pallas import tpu_sc as plsc

D = 64
B = 16384

NCORES = 1
NS = 16
NW = NCORES * NS
B_PER_W = B // NW
CHUNK = 128
NCHUNK = B_PER_W // CHUNK
NSLOT = 2
GROUPS_PER_CHUNK = CHUNK // 16
ROWS2D = B // CHUNK

_LN2 = 0.6931471805599453
_SQRT_MAGIC = 0x1FBD1DF5


def _sc_kernel_body(user_table, item_table, user_bias, item_bias, gb16,
                    uids2d, iids2d, out,
                    uidx_v, iidx_v, u_rows, v_rows, ub_v, ib_v,
                    o_v, gb_v, s0, s1):
    if NCORES == 1:
        wid = lax.axis_index("s")
    else:
        wid = lax.axis_index("s") * NCORES + lax.axis_index("c")
    row0 = wid * NCHUNK
    sems = [s0, s1]

    pltpu.sync_copy(gb16, gb_v)
    pltpu.sync_copy(uids2d.at[pl.ds(row0, NCHUNK)], uidx_v)
    pltpu.sync_copy(iids2d.at[pl.ds(row0, NCHUNK)], iidx_v)

    def fire(c):
        slot = c % NSLOT
        sem = sems[slot]
        return (
            pltpu.async_copy(user_table.at[uidx_v.at[c]],
                             u_rows.at[pl.ds(slot * CHUNK, CHUNK)], sem),
            pltpu.async_copy(item_table.at[iidx_v.at[c]],
                             v_rows.at[pl.ds(slot * CHUNK, CHUNK)], sem),
            pltpu.async_copy(user_bias.at[uidx_v.at[c]], ub_v.at[c], sem),
            pltpu.async_copy(item_bias.at[iidx_v.at[c]], ib_v.at[c], sem),
        )

    lanes = lax.iota(jnp.int32, 16)
    ix8 = lax.bitwise_xor(lanes, 8)
    ix4 = lax.bitwise_xor(lanes, 4)
    ix2 = lax.bitwise_xor(lanes, 2)
    ix1 = lax.bitwise_xor(lanes, 1)
    in_lo4 = lanes < 4
    in_lo8 = lanes < 8
    gb = gb_v[...]

    def shuf(v, ix):
        return v.at[ix].get(mode="promise_in_bounds")

    def f32bits(v):
        return lax.bitcast_convert_type(v, jnp.int32)

    def bits32f(v):
        return lax.bitcast_convert_type(v, jnp.float32)

    handles = [fire(0), fire(1)]

    for c in range(NCHUNK):
        for h in handles[c % NSLOT]:
            h.wait()
        slot = c % NSLOT

        def group_body(j, _, c=c, slot=slot):
            uu = jnp.zeros((16,), jnp.float32)
            vv = jnp.zeros((16,), jnp.float32)
            dd = jnp.zeros((16,), jnp.float32)
            for jj in range(16):
                r = slot * CHUNK + j * 16 + jj
                uu_p = jnp.zeros((16,), jnp.float32)
                vv_p = jnp.zeros((16,), jnp.float32)
                dd_p = jnp.zeros((16,), jnp.float32)
                for k in range(D // 16):
                    uk = u_rows[r, pl.ds(k * 16, 16)]
                    vk = v_rows[r, pl.ds(k * 16, 16)]
                    uu_p = uu_p + uk * uk
                    vv_p = vv_p + vk * vk
                    dk = uk - vk
                    dd_p = dd_p + dk * dk
                uu_p = uu_p + shuf(uu_p, ix8)
                vv_p = vv_p + shuf(vv_p, ix8)
                dd_p = dd_p + shuf(dd_p, ix8)
                uu_p = uu_p + shuf(uu_p, ix4)
                vv_p = vv_p + shuf(vv_p, ix4)
                dd_p = dd_p + shuf(dd_p, ix4)
                w = jnp.where(in_lo4, uu_p, jnp.where(in_lo8, vv_p, dd_p))
                w = w + shuf(w, ix2)
                w = w + shuf(w, ix1)
                m = lanes == jj
                uu = jnp.where(m, w[0], uu)
                vv = jnp.where(m, w[4], vv)
                dd = jnp.where(m, w[8], dd)
            den = (1.0 - uu) * (1.0 - vv) + 1e-6
            t = (2.0 * dd) / den
            x = 1.0 + t
            y = t * (t + 2.0)
            g = bits32f(lax.shift_right_logical(f32bits(y), 1)
                        + _SQRT_MAGIC)
            for _newton in range(4):
                g = 0.5 * (g + y / g)
            z = x + g
            zb = f32bits(z)
            e = lax.convert_element_type(
                lax.shift_right_logical(zb, 23) - 127, jnp.float32)
            mant = bits32f(lax.bitwise_or(
                lax.bitwise_and(zb, 0x007FFFFF), 0x3F800000))
            wq = (mant - 1.0) / (mant + 1.0)
            w2 = wq * wq
            p = jnp.full((16,), 1.0 / 9.0, jnp.float32)
            for coef in (1.0 / 7.0, 1.0 / 5.0, 1.0 / 3.0, 1.0):
                p = p * w2 + coef
            dist = e * _LN2 + 2.0 * wq * p
            ubv = ub_v[c, pl.ds(j * 16, 16)]
            ibv = ib_v[c, pl.ds(j * 16, 16)]
            o_v[c, pl.ds(j * 16, 16)] = ubv + ibv + gb - dist
            return _

        lax.fori_loop(0, GROUPS_PER_CHUNK, group_body, None)
        if c + NSLOT < NCHUNK:
            handles[c % NSLOT] = fire(c + NSLOT)
        pltpu.sync_copy(o_v.at[pl.ds(c, 1)], out.at[pl.ds(row0 + c, 1)])


@jax.jit
def _sc_stage(uids2d, iids2d, user_table, item_table, user_bias, item_bias,
              gb16):
    mesh = plsc.VectorSubcoreMesh(core_axis_name="c", subcore_axis_name="s",
                                  num_cores=NCORES)
    f = functools.partial(
        pl.kernel,
        mesh=mesh,
        out_type=jax.ShapeDtypeStruct((ROWS2D, CHUNK), jnp.float32),
        scratch_types=[
            pltpu.VMEM((NCHUNK, CHUNK), jnp.int32),
            pltpu.VMEM((NCHUNK, CHUNK), jnp.int32),
            pltpu.VMEM((NSLOT * CHUNK, D), jnp.float32),
            pltpu.VMEM((NSLOT * CHUNK, D), jnp.float32),
            pltpu.VMEM((NCHUNK, CHUNK), jnp.float32),
            pltpu.VMEM((NCHUNK, CHUNK), jnp.float32),
            pltpu.VMEM((NCHUNK, CHUNK), jnp.float32),
            pltpu.VMEM((16,), jnp.float32),
            pltpu.SemaphoreType.DMA,
            pltpu.SemaphoreType.DMA,
        ],
        compiler_params=pltpu.CompilerParams(use_tc_tiling_on_sc=False),
    )(_sc_kernel_body)
    return f(user_table, item_table, user_bias, item_bias, gb16,
             uids2d, iids2d)


@jax.jit
def kernel(user_ids, item_ids, user_table, item_table, user_bias_table,
           item_bias_table, global_bias):
    uids2d = user_ids.reshape(ROWS2D, CHUNK)
    iids2d = item_ids.reshape(ROWS2D, CHUNK)
    gb16 = jnp.broadcast_to(global_bias, (16,))
    out2d = _sc_stage(uids2d, iids2d, user_table, item_table,
                      user_bias_table.reshape(-1),
                      item_bias_table.reshape(-1), gb16)
    return out2d.reshape(B)
